# baseline (speedup 1.0000x reference)
import functools

import jax
import jax.numpy as jnp
from jax import lax
from jax.experimental import pallas as pl
from jax.experimental.pallas import tpu as pltpu

EPS_ = 1e-10
TILE_R = 128


def _k1_kernel(row_ref, col_ref, rm_ref, b_ref, num_ref, den_ref, num_acc, den_acc,
               *, h, w, nb, nt):
    b = pl.program_id(0)
    t = pl.program_id(1)

    @pl.when(jnp.logical_and(b == 0, t == 0))
    def _init():
        num_acc[0] = 0.0
        den_acc[0] = 0.0

    x = rm_ref[0]
    bmap = b_ref[0, 0]
    row = row_ref[0, 0]
    col = col_ref[0, 0]

    iota_w = lax.broadcasted_iota(jnp.int32, (TILE_R, 1, w), 2)
    mask_w = iota_w == col[:, None, None]
    iota_h = lax.broadcasted_iota(jnp.int32, (TILE_R, h), 1)
    mask_h = iota_h == row[:, None]

    sum_w = jnp.sum(x, axis=2)
    denom = jnp.sum(sum_w, axis=1)

    srm_w = jnp.sum(jnp.where(mask_w, x, 0.0), axis=2)
    srm = jnp.sum(jnp.where(mask_h, srm_w, 0.0), axis=1)

    sb_w = jnp.sum(jnp.where(mask_w, bmap[None], 0.0), axis=2)
    sb = jnp.sum(jnp.where(mask_h, sb_w, 0.0), axis=1)

    num_acc[0] += jnp.sum(sb * -jnp.log(EPS_ + srm / denom))
    den_acc[0] += jnp.sum(sb)

    @pl.when(jnp.logical_and(b == nb - 1, t == nt - 1))
    def _fin():
        num_ref[...] = jnp.full((1, 1), num_acc[0], jnp.float32)
        den_ref[...] = jnp.full((1, 1), den_acc[0], jnp.float32)


def _k2_kernel(loc_ref, row_ref, col_ref, xf_ref, b_ref, num_in, den_in,
               out_ref, num_acc, den_acc, *, h, w, hw, nb, nt):
    b = pl.program_id(0)
    t = pl.program_id(1)

    @pl.when(jnp.logical_and(b == 0, t == 0))
    def _init():
        num_acc[0] = num_in[0, 0]
        den_acc[0] = den_in[0, 0]

    x = xf_ref[...]
    bmap = b_ref[0, 0]
    loc = loc_ref[0, 0]
    row = row_ref[0, 0]
    col = col_ref[0, 0]

    iota_f = lax.broadcasted_iota(jnp.int32, (TILE_R, hw), 1)
    mask_f = iota_f == loc[:, None]

    denom = jnp.sum(x, axis=1)
    srm = jnp.sum(jnp.where(mask_f, x, 0.0), axis=1)

    iota_w = lax.broadcasted_iota(jnp.int32, (TILE_R, 1, w), 2)
    mask_w = iota_w == col[:, None, None]
    iota_h = lax.broadcasted_iota(jnp.int32, (TILE_R, h), 1)
    mask_h = iota_h == row[:, None]
    sb_w = jnp.sum(jnp.where(mask_w, bmap[None], 0.0), axis=2)
    sb = jnp.sum(jnp.where(mask_h, sb_w, 0.0), axis=1)

    num_acc[0] += jnp.sum(sb * -jnp.log(EPS_ + srm / denom))
    den_acc[0] += jnp.sum(sb)

    @pl.when(jnp.logical_and(b == nb - 1, t == nt - 1))
    def _fin():
        out_ref[...] = jnp.full((1, 1), num_acc[0] / (1.0 + den_acc[0]), jnp.float32)


def kernel(response_map, source_feature_1d_locations, boundaries):
    B, S, H, W = response_map.shape
    HW = H * W
    B1 = B // 2
    B2 = B - B1
    T = S // TILE_R

    loc = source_feature_1d_locations.astype(jnp.int32)
    row = loc // W
    col = loc % W

    row1 = row[:B1].reshape(B1 * T, 1, TILE_R)
    col1 = col[:B1].reshape(B1 * T, 1, TILE_R)
    loc2 = loc[B1:].reshape(B2 * T, 1, TILE_R)
    row2 = row[B1:].reshape(B2 * T, 1, TILE_R)
    col2 = col[B1:].reshape(B2 * T, 1, TILE_R)

    rm_flat2 = response_map[B1:].reshape(B2 * S, HW)

    num1, den1 = pl.pallas_call(
        functools.partial(_k1_kernel, h=H, w=W, nb=B1, nt=T),
        grid=(B1, T),
        in_specs=[
            pl.BlockSpec((1, 1, TILE_R), lambda b, t: (b * T + t, 0, 0)),
            pl.BlockSpec((1, 1, TILE_R), lambda b, t: (b * T + t, 0, 0)),
            pl.BlockSpec((1, TILE_R, H, W), lambda b, t: (b, t, 0, 0)),
            pl.BlockSpec((1, 1, H, W), lambda b, t: (b, 0, 0, 0)),
        ],
        out_specs=[
            pl.BlockSpec((1, 1), lambda b, t: (0, 0)),
            pl.BlockSpec((1, 1), lambda b, t: (0, 0)),
        ],
        out_shape=[
            jax.ShapeDtypeStruct((1, 1), jnp.float32),
            jax.ShapeDtypeStruct((1, 1), jnp.float32),
        ],
        scratch_shapes=[
            pltpu.SMEM((1,), jnp.float32),
            pltpu.SMEM((1,), jnp.float32),
        ],
    )(row1, col1, response_map[:B1], boundaries[:B1])

    out = pl.pallas_call(
        functools.partial(_k2_kernel, h=H, w=W, hw=HW, nb=B2, nt=T),
        grid=(B2, T),
        in_specs=[
            pl.BlockSpec((1, 1, TILE_R), lambda b, t: (b * T + t, 0, 0)),
            pl.BlockSpec((1, 1, TILE_R), lambda b, t: (b * T + t, 0, 0)),
            pl.BlockSpec((1, 1, TILE_R), lambda b, t: (b * T + t, 0, 0)),
            pl.BlockSpec((TILE_R, HW), lambda b, t: (b * T + t, 0)),
            pl.BlockSpec((1, 1, H, W), lambda b, t: (b, 0, 0, 0)),
            pl.BlockSpec((1, 1), lambda b, t: (0, 0)),
            pl.BlockSpec((1, 1), lambda b, t: (0, 0)),
        ],
        out_specs=pl.BlockSpec((1, 1), lambda b, t: (0, 0)),
        out_shape=jax.ShapeDtypeStruct((1, 1), jnp.float32),
        scratch_shapes=[
            pltpu.SMEM((1,), jnp.float32),
            pltpu.SMEM((1,), jnp.float32),
        ],
    )(loc2, row2, col2, rm_flat2, boundaries[B1:], num1, den1)
    return out[0, 0]
